# baseline (speedup 1.0000x reference)
import functools

import jax
import jax.numpy as jnp
from jax import lax
from jax.experimental import pallas as pl
from jax.experimental.pallas import tpu as pltpu
from jax.experimental.pallas import tpu_sc as plsc

_NUM_CODES = 8192
_CODE_DIM = 256
_M = 9216


_PROJ_BLK = 2048


def _proj_body(cb_ref, pw_ref, pb_ref, qc_ref):
    qc_ref[...] = lax.dot_general(
        cb_ref[...], pw_ref[...], (((1,), (1,)), ((), ())),
        preferred_element_type=jnp.float32) + pb_ref[...]


def _project(codebook_w, proj_w, proj_b2d):
    return pl.pallas_call(
        _proj_body,
        grid=(_NUM_CODES // _PROJ_BLK,),
        in_specs=[
            pl.BlockSpec((_PROJ_BLK, _CODE_DIM), lambda i: (i, 0)),
            pl.BlockSpec((_CODE_DIM, _CODE_DIM), lambda i: (0, 0)),
            pl.BlockSpec((1, _CODE_DIM), lambda i: (0, 0)),
        ],
        out_specs=pl.BlockSpec((_PROJ_BLK, _CODE_DIM), lambda i: (i, 0)),
        out_shape=jax.ShapeDtypeStruct((_NUM_CODES, _CODE_DIM), jnp.float32),
        compiler_params=pltpu.CompilerParams(
            dimension_semantics=("parallel",)),
    )(codebook_w, proj_w, proj_b2d)


_ZBLK = 256


def _argmin_body(z_ref, qc_ref, zn_ref, cn_ref, io_ref, idx_ref):
    s2 = lax.dot_general(
        z_ref[...] * -2.0, qc_ref[...], (((1,), (1,)), ((), ())),
        preferred_element_type=jnp.float32)
    d = (zn_ref[...] + cn_ref[...]) + s2
    bmin = jnp.min(d, axis=1, keepdims=True)
    idx_ref[...] = jnp.min(
        jnp.where(d == bmin, io_ref[...], jnp.float32(3.0e38)),
        axis=1, keepdims=True).astype(jnp.int32)


def _argmin_codes(z2d, qc, znorm, cnorm_row):
    iota_row = lax.iota(jnp.float32, _NUM_CODES)[None, :]
    return pl.pallas_call(
        _argmin_body,
        grid=(_M // _ZBLK,),
        in_specs=[
            pl.BlockSpec((_ZBLK, _CODE_DIM), lambda i: (i, 0)),
            pl.BlockSpec((_NUM_CODES, _CODE_DIM), lambda i: (0, 0)),
            pl.BlockSpec((_ZBLK, 1), lambda i: (i, 0)),
            pl.BlockSpec((1, _NUM_CODES), lambda i: (0, 0)),
            pl.BlockSpec((1, _NUM_CODES), lambda i: (0, 0)),
        ],
        out_specs=pl.BlockSpec((_ZBLK, 1), lambda i: (i, 0)),
        out_shape=jax.ShapeDtypeStruct((_M, 1), jnp.int32),
        compiler_params=pltpu.CompilerParams(
            dimension_semantics=("parallel",)),
    )(z2d, qc, znorm, cnorm_row, iota_row)


_NC, _NS = 2, 16
_NW = _NC * _NS
_BPW = _M // _NW
_CHUNK = 96
_NCHUNK = _BPW // _CHUNK


def _gather_body(table_hbm, idx_hbm, out_hbm, i0, i1, i2, rows_v, sem):
    wid = lax.axis_index("c") * _NS + lax.axis_index("s")
    base = wid * _BPW
    bufs = (i0, i1, i2)
    for c in range(_NCHUNK):
        pltpu.sync_copy(idx_hbm.at[pl.ds(base + c * _CHUNK, _CHUNK)], bufs[c])
    cps = [
        pltpu.async_copy(table_hbm.at[bufs[c]],
                         rows_v.at[pl.ds(c * _CHUNK, _CHUNK)], sem)
        for c in range(_NCHUNK)
    ]
    for cp in cps:
        cp.wait()
    pltpu.sync_copy(rows_v, out_hbm.at[pl.ds(base, _BPW)])


def _gather_rows(qc, idx_flat):
    mesh = plsc.VectorSubcoreMesh(core_axis_name="c", subcore_axis_name="s")
    f = pl.kernel(
        _gather_body,
        out_type=jax.ShapeDtypeStruct((_M, _CODE_DIM), jnp.float32),
        mesh=mesh,
        scratch_types=[
            pltpu.VMEM((_CHUNK,), jnp.int32),
            pltpu.VMEM((_CHUNK,), jnp.int32),
            pltpu.VMEM((_CHUNK,), jnp.int32),
            pltpu.VMEM((_BPW, _CODE_DIM), jnp.float32),
            pltpu.SemaphoreType.DMA,
        ],
    )
    return f(qc, idx_flat)


def kernel(z, codebook_w, proj_w, proj_b):
    z2d = z.reshape(-1, _CODE_DIM)
    qc = _project(codebook_w, proj_w, proj_b.reshape(1, _CODE_DIM))
    znorm = jnp.sum(z2d ** 2, axis=1, keepdims=True)
    cnorm_row = jnp.sum(qc ** 2, axis=1)[None, :]
    idx = _argmin_codes(z2d, qc, znorm, cnorm_row).reshape(-1)
    z_q = _gather_rows(qc, idx)
    return z_q.reshape(z.shape), idx.reshape(z.shape[:-1])


# speedup vs baseline: 1.1883x; 1.1883x over previous
import functools

import jax
import jax.numpy as jnp
from jax import lax
from jax.experimental import pallas as pl
from jax.experimental.pallas import tpu as pltpu
from jax.experimental.pallas import tpu_sc as plsc

_NUM_CODES = 8192
_CODE_DIM = 256
_M = 9216


_PROJ_BLK = 2048


def _proj_body(cb_ref, pw_ref, pb_ref, qc_ref):
    qc_ref[...] = lax.dot_general(
        cb_ref[...], pw_ref[...], (((1,), (1,)), ((), ())),
        preferred_element_type=jnp.float32) + pb_ref[...]


def _project(codebook_w, proj_w, proj_b2d):
    return pl.pallas_call(
        _proj_body,
        grid=(_NUM_CODES // _PROJ_BLK,),
        in_specs=[
            pl.BlockSpec((_PROJ_BLK, _CODE_DIM), lambda i: (i, 0)),
            pl.BlockSpec((_CODE_DIM, _CODE_DIM), lambda i: (0, 0)),
            pl.BlockSpec((1, _CODE_DIM), lambda i: (0, 0)),
        ],
        out_specs=pl.BlockSpec((_PROJ_BLK, _CODE_DIM), lambda i: (i, 0)),
        out_shape=jax.ShapeDtypeStruct((_NUM_CODES, _CODE_DIM), jnp.float32),
        compiler_params=pltpu.CompilerParams(
            dimension_semantics=("parallel",)),
    )(codebook_w, proj_w, proj_b2d)


_ZBLK = 256


_LANES = 128
_NCHUNKS = _NUM_CODES // _LANES


def _argmin_body(z_ref, qc_ref, zn_ref, cn_ref, idx_ref):
    s2 = lax.dot_general(
        z_ref[...] * -2.0, qc_ref[...], (((1,), (1,)), ((), ())),
        preferred_element_type=jnp.float32)
    zn = zn_ref[...]
    cn = cn_ref[...]
    m_acc = jnp.full((_ZBLK, _LANES), jnp.float32(3.0e38))
    a_acc = jnp.zeros((_ZBLK, _LANES), jnp.int32)
    for g in range(_NCHUNKS):
        dg = (zn + cn[:, g * _LANES:(g + 1) * _LANES]) \
            + s2[:, g * _LANES:(g + 1) * _LANES]
        upd = dg < m_acc
        a_acc = jnp.where(upd, jnp.int32(g), a_acc)
        m_acc = jnp.minimum(m_acc, dg)
    bmin = jnp.min(m_acc, axis=1, keepdims=True)
    lane = lax.broadcasted_iota(jnp.int32, (_ZBLK, _LANES), 1)
    j = a_acc * _LANES + lane
    idx_ref[...] = jnp.min(
        jnp.where(m_acc == bmin, j, jnp.int32(1 << 30)),
        axis=1, keepdims=True)


def _argmin_codes(z2d, qc, znorm, cnorm_row):
    return pl.pallas_call(
        _argmin_body,
        grid=(_M // _ZBLK,),
        in_specs=[
            pl.BlockSpec((_ZBLK, _CODE_DIM), lambda i: (i, 0)),
            pl.BlockSpec((_NUM_CODES, _CODE_DIM), lambda i: (0, 0)),
            pl.BlockSpec((_ZBLK, 1), lambda i: (i, 0)),
            pl.BlockSpec((1, _NUM_CODES), lambda i: (0, 0)),
        ],
        out_specs=pl.BlockSpec((_ZBLK, 1), lambda i: (i, 0)),
        out_shape=jax.ShapeDtypeStruct((_M, 1), jnp.int32),
        compiler_params=pltpu.CompilerParams(
            dimension_semantics=("parallel",)),
    )(z2d, qc, znorm, cnorm_row)


_NC, _NS = 2, 16
_NW = _NC * _NS
_BPW = _M // _NW
_CHUNK = 96
_NCHUNK = _BPW // _CHUNK


def _gather_body(table_hbm, idx_hbm, out_hbm, i0, i1, i2, rows_v, sem):
    wid = lax.axis_index("c") * _NS + lax.axis_index("s")
    base = wid * _BPW
    bufs = (i0, i1, i2)
    for c in range(_NCHUNK):
        pltpu.sync_copy(idx_hbm.at[pl.ds(base + c * _CHUNK, _CHUNK)], bufs[c])
    cps = [
        pltpu.async_copy(table_hbm.at[bufs[c]],
                         rows_v.at[pl.ds(c * _CHUNK, _CHUNK)], sem)
        for c in range(_NCHUNK)
    ]
    for cp in cps:
        cp.wait()
    pltpu.sync_copy(rows_v, out_hbm.at[pl.ds(base, _BPW)])


def _gather_rows(qc, idx_flat):
    mesh = plsc.VectorSubcoreMesh(core_axis_name="c", subcore_axis_name="s")
    f = pl.kernel(
        _gather_body,
        out_type=jax.ShapeDtypeStruct((_M, _CODE_DIM), jnp.float32),
        mesh=mesh,
        scratch_types=[
            pltpu.VMEM((_CHUNK,), jnp.int32),
            pltpu.VMEM((_CHUNK,), jnp.int32),
            pltpu.VMEM((_CHUNK,), jnp.int32),
            pltpu.VMEM((_BPW, _CODE_DIM), jnp.float32),
            pltpu.SemaphoreType.DMA,
        ],
    )
    return f(qc, idx_flat)


def kernel(z, codebook_w, proj_w, proj_b):
    z2d = z.reshape(-1, _CODE_DIM)
    qc = _project(codebook_w, proj_w, proj_b.reshape(1, _CODE_DIM))
    znorm = jnp.sum(z2d ** 2, axis=1, keepdims=True)
    cnorm_row = jnp.sum(qc ** 2, axis=1)[None, :]
    idx = _argmin_codes(z2d, qc, znorm, cnorm_row).reshape(-1)
    z_q = _gather_rows(qc, idx)
    return z_q.reshape(z.shape), idx.reshape(z.shape[:-1])


# speedup vs baseline: 1.3114x; 1.1036x over previous
import functools

import jax
import jax.numpy as jnp
from jax import lax
from jax.experimental import pallas as pl
from jax.experimental.pallas import tpu as pltpu
from jax.experimental.pallas import tpu_sc as plsc

_NUM_CODES = 8192
_CODE_DIM = 256
_M = 9216


_PROJ_BLK = 2048


def _proj_body(cb_ref, pw_ref, pb_ref, qc_ref):
    qc_ref[...] = lax.dot_general(
        cb_ref[...], pw_ref[...], (((1,), (1,)), ((), ())),
        preferred_element_type=jnp.float32) + pb_ref[...]


def _project(codebook_w, proj_w, proj_b2d):
    return pl.pallas_call(
        _proj_body,
        grid=(_NUM_CODES // _PROJ_BLK,),
        in_specs=[
            pl.BlockSpec((_PROJ_BLK, _CODE_DIM), lambda i: (i, 0)),
            pl.BlockSpec((_CODE_DIM, _CODE_DIM), lambda i: (0, 0)),
            pl.BlockSpec((1, _CODE_DIM), lambda i: (0, 0)),
        ],
        out_specs=pl.BlockSpec((_PROJ_BLK, _CODE_DIM), lambda i: (i, 0)),
        out_shape=jax.ShapeDtypeStruct((_NUM_CODES, _CODE_DIM), jnp.float32),
        compiler_params=pltpu.CompilerParams(
            dimension_semantics=("parallel",)),
    )(codebook_w, proj_w, proj_b2d)


_ZBLK = 256


_LANES = 128
_NCHUNKS = _NUM_CODES // _LANES


def _argmin_body(z_ref, qc_ref, cn_ref, idx_ref):
    z = z_ref[...]
    s2 = lax.dot_general(
        z * -2.0, qc_ref[...], (((1,), (1,)), ((), ())),
        preferred_element_type=jnp.float32)
    zn = jnp.sum(z * z, axis=1, keepdims=True)
    cn = cn_ref[...]
    m_acc = jnp.full((_ZBLK, _LANES), jnp.float32(3.0e38))
    a_acc = jnp.zeros((_ZBLK, _LANES), jnp.int32)
    for g in range(_NCHUNKS):
        dg = (zn + cn[:, g * _LANES:(g + 1) * _LANES]) \
            + s2[:, g * _LANES:(g + 1) * _LANES]
        upd = dg < m_acc
        a_acc = jnp.where(upd, jnp.int32(g), a_acc)
        m_acc = jnp.minimum(m_acc, dg)
    bmin = jnp.min(m_acc, axis=1, keepdims=True)
    lane = lax.broadcasted_iota(jnp.int32, (_ZBLK, _LANES), 1)
    j = a_acc * _LANES + lane
    idx_ref[...] = jnp.min(
        jnp.where(m_acc == bmin, j, jnp.int32(1 << 30)),
        axis=1, keepdims=True)


def _argmin_codes(z2d, qc, cnorm_row):
    return pl.pallas_call(
        _argmin_body,
        grid=(_M // _ZBLK,),
        in_specs=[
            pl.BlockSpec((_ZBLK, _CODE_DIM), lambda i: (i, 0)),
            pl.BlockSpec((_NUM_CODES, _CODE_DIM), lambda i: (0, 0)),
            pl.BlockSpec((1, _NUM_CODES), lambda i: (0, 0)),
        ],
        out_specs=pl.BlockSpec((_ZBLK, 1), lambda i: (i, 0)),
        out_shape=jax.ShapeDtypeStruct((_M, 1), jnp.int32),
        compiler_params=pltpu.CompilerParams(
            dimension_semantics=("parallel",)),
    )(z2d, qc, cnorm_row)


_NC, _NS = 2, 16
_NW = _NC * _NS
_BPW = _M // _NW
_CHUNK = 96
_NCHUNK = _BPW // _CHUNK


def _gather_body(table_hbm, idx_hbm, out_hbm, i0, i1, i2, rows_v, sem):
    wid = lax.axis_index("c") * _NS + lax.axis_index("s")
    base = wid * _BPW
    bufs = (i0, i1, i2)
    for c in range(_NCHUNK):
        pltpu.sync_copy(idx_hbm.at[pl.ds(base + c * _CHUNK, _CHUNK)], bufs[c])
    cps = [
        pltpu.async_copy(table_hbm.at[bufs[c]],
                         rows_v.at[pl.ds(c * _CHUNK, _CHUNK)], sem)
        for c in range(_NCHUNK)
    ]
    for cp in cps:
        cp.wait()
    pltpu.sync_copy(rows_v, out_hbm.at[pl.ds(base, _BPW)])


def _gather_rows(qc, idx_flat):
    mesh = plsc.VectorSubcoreMesh(core_axis_name="c", subcore_axis_name="s")
    f = pl.kernel(
        _gather_body,
        out_type=jax.ShapeDtypeStruct((_M, _CODE_DIM), jnp.float32),
        mesh=mesh,
        scratch_types=[
            pltpu.VMEM((_CHUNK,), jnp.int32),
            pltpu.VMEM((_CHUNK,), jnp.int32),
            pltpu.VMEM((_CHUNK,), jnp.int32),
            pltpu.VMEM((_BPW, _CODE_DIM), jnp.float32),
            pltpu.SemaphoreType.DMA,
        ],
    )
    return f(qc, idx_flat)


def kernel(z, codebook_w, proj_w, proj_b):
    z2d = z.reshape(-1, _CODE_DIM)
    qc = _project(codebook_w, proj_w, proj_b.reshape(1, _CODE_DIM))
    cnorm_row = jnp.sum(qc ** 2, axis=1)[None, :]
    idx = _argmin_codes(z2d, qc, cnorm_row).reshape(-1)
    z_q = _gather_rows(qc, idx)
    return z_q.reshape(z.shape), idx.reshape(z.shape[:-1])


# speedup vs baseline: 1.3765x; 1.0496x over previous
import functools

import jax
import jax.numpy as jnp
from jax import lax
from jax.experimental import pallas as pl
from jax.experimental.pallas import tpu as pltpu
from jax.experimental.pallas import tpu_sc as plsc

_NUM_CODES = 8192
_CODE_DIM = 256
_M = 9216


_ZBLK = 256


_LANES = 128
_NCHUNKS = _NUM_CODES // _LANES


def _argmin_body(z_ref, cb_ref, pw_ref, pb_ref, idx_ref, qc_ref, cn_ref):
    @pl.when(pl.program_id(0) == 0)
    def _():
        qc = lax.dot_general(
            cb_ref[...], pw_ref[...], (((1,), (1,)), ((), ())),
            preferred_element_type=jnp.float32) + pb_ref[...]
        qc_ref[...] = qc
        cn_ref[...] = jnp.sum(qc * qc, axis=1, keepdims=True).reshape(
            1, _NUM_CODES)

    z = z_ref[...]
    s2 = lax.dot_general(
        z * -2.0, qc_ref[...], (((1,), (1,)), ((), ())),
        preferred_element_type=jnp.float32)
    zn = jnp.sum(z * z, axis=1, keepdims=True)
    cn = cn_ref[...]
    m_acc = jnp.full((_ZBLK, _LANES), jnp.float32(3.0e38))
    a_acc = jnp.zeros((_ZBLK, _LANES), jnp.int32)
    for g in range(_NCHUNKS):
        dg = (zn + cn[:, g * _LANES:(g + 1) * _LANES]) \
            + s2[:, g * _LANES:(g + 1) * _LANES]
        upd = dg < m_acc
        a_acc = jnp.where(upd, jnp.int32(g), a_acc)
        m_acc = jnp.minimum(m_acc, dg)
    bmin = jnp.min(m_acc, axis=1, keepdims=True)
    lane = lax.broadcasted_iota(jnp.int32, (_ZBLK, _LANES), 1)
    j = a_acc * _LANES + lane
    idx_ref[...] = jnp.min(
        jnp.where(m_acc == bmin, j, jnp.int32(1 << 30)),
        axis=1, keepdims=True)


def _argmin_codes(z2d, codebook_w, proj_w, proj_b2d):
    return pl.pallas_call(
        _argmin_body,
        grid=(_M // _ZBLK,),
        in_specs=[
            pl.BlockSpec((_ZBLK, _CODE_DIM), lambda i: (i, 0)),
            pl.BlockSpec((_NUM_CODES, _CODE_DIM), lambda i: (0, 0)),
            pl.BlockSpec((_CODE_DIM, _CODE_DIM), lambda i: (0, 0)),
            pl.BlockSpec((1, _CODE_DIM), lambda i: (0, 0)),
        ],
        out_specs=[
            pl.BlockSpec((_ZBLK, 1), lambda i: (i, 0)),
            pl.BlockSpec((_NUM_CODES, _CODE_DIM), lambda i: (0, 0)),
        ],
        out_shape=[
            jax.ShapeDtypeStruct((_M, 1), jnp.int32),
            jax.ShapeDtypeStruct((_NUM_CODES, _CODE_DIM), jnp.float32),
        ],
        scratch_shapes=[pltpu.VMEM((1, _NUM_CODES), jnp.float32)],
    )(z2d, codebook_w, proj_w, proj_b2d)


_NC, _NS = 2, 16
_NW = _NC * _NS
_BPW = _M // _NW
_CHUNK = 96
_NCHUNK = _BPW // _CHUNK


def _gather_body(table_hbm, idx_hbm, out_hbm, i0, i1, i2, rows_v, sem):
    wid = lax.axis_index("c") * _NS + lax.axis_index("s")
    base = wid * _BPW
    bufs = (i0, i1, i2)
    for c in range(_NCHUNK):
        pltpu.sync_copy(idx_hbm.at[pl.ds(base + c * _CHUNK, _CHUNK)], bufs[c])
    cps = [
        pltpu.async_copy(table_hbm.at[bufs[c]],
                         rows_v.at[pl.ds(c * _CHUNK, _CHUNK)], sem)
        for c in range(_NCHUNK)
    ]
    for cp in cps:
        cp.wait()
    pltpu.sync_copy(rows_v, out_hbm.at[pl.ds(base, _BPW)])


def _gather_rows(qc, idx_flat):
    mesh = plsc.VectorSubcoreMesh(core_axis_name="c", subcore_axis_name="s")
    f = pl.kernel(
        _gather_body,
        out_type=jax.ShapeDtypeStruct((_M, _CODE_DIM), jnp.float32),
        mesh=mesh,
        scratch_types=[
            pltpu.VMEM((_CHUNK,), jnp.int32),
            pltpu.VMEM((_CHUNK,), jnp.int32),
            pltpu.VMEM((_CHUNK,), jnp.int32),
            pltpu.VMEM((_BPW, _CODE_DIM), jnp.float32),
            pltpu.SemaphoreType.DMA,
        ],
    )
    return f(qc, idx_flat)


def kernel(z, codebook_w, proj_w, proj_b):
    z2d = z.reshape(-1, _CODE_DIM)
    idx2d, qc = _argmin_codes(
        z2d, codebook_w, proj_w, proj_b.reshape(1, _CODE_DIM))
    idx = idx2d.reshape(-1)
    z_q = _gather_rows(qc, idx)
    return z_q.reshape(z.shape), idx.reshape(z.shape[:-1])


# speedup vs baseline: 1.4707x; 1.0684x over previous
import functools

import jax
import jax.numpy as jnp
from jax import lax
from jax.experimental import pallas as pl
from jax.experimental.pallas import tpu as pltpu
from jax.experimental.pallas import tpu_sc as plsc

_NUM_CODES = 8192
_CODE_DIM = 256
_M = 9216


_ZBLK = 512


_LANES = 128
_NCHUNKS = _NUM_CODES // _LANES


def _argmin_body(z_ref, cb_ref, pw_ref, pb_ref, idx_ref, qc_ref, cn_ref):
    @pl.when(pl.program_id(0) == 0)
    def _():
        qc = lax.dot_general(
            cb_ref[...], pw_ref[...], (((1,), (1,)), ((), ())),
            preferred_element_type=jnp.float32) + pb_ref[...]
        qc_ref[...] = qc
        cn_ref[...] = jnp.sum(qc * qc, axis=1, keepdims=True).reshape(
            1, _NUM_CODES)

    z = z_ref[...]
    s2 = lax.dot_general(
        z * -2.0, qc_ref[...], (((1,), (1,)), ((), ())),
        preferred_element_type=jnp.float32)
    zn = jnp.sum(z * z, axis=1, keepdims=True)
    cn = cn_ref[...]
    m_acc = jnp.full((_ZBLK, _LANES), jnp.float32(3.0e38))
    a_acc = jnp.zeros((_ZBLK, _LANES), jnp.int32)
    for g in range(_NCHUNKS):
        dg = (zn + cn[:, g * _LANES:(g + 1) * _LANES]) \
            + s2[:, g * _LANES:(g + 1) * _LANES]
        upd = dg < m_acc
        a_acc = jnp.where(upd, jnp.int32(g), a_acc)
        m_acc = jnp.minimum(m_acc, dg)
    bmin = jnp.min(m_acc, axis=1, keepdims=True)
    lane = lax.broadcasted_iota(jnp.int32, (_ZBLK, _LANES), 1)
    j = a_acc * _LANES + lane
    idx_ref[...] = jnp.min(
        jnp.where(m_acc == bmin, j, jnp.int32(1 << 30)),
        axis=1, keepdims=True)


def _argmin_codes(z2d, codebook_w, proj_w, proj_b2d):
    return pl.pallas_call(
        _argmin_body,
        grid=(_M // _ZBLK,),
        in_specs=[
            pl.BlockSpec((_ZBLK, _CODE_DIM), lambda i: (i, 0)),
            pl.BlockSpec((_NUM_CODES, _CODE_DIM), lambda i: (0, 0)),
            pl.BlockSpec((_CODE_DIM, _CODE_DIM), lambda i: (0, 0)),
            pl.BlockSpec((1, _CODE_DIM), lambda i: (0, 0)),
        ],
        out_specs=[
            pl.BlockSpec((_ZBLK, 1), lambda i: (i, 0)),
            pl.BlockSpec((_NUM_CODES, _CODE_DIM), lambda i: (0, 0)),
        ],
        out_shape=[
            jax.ShapeDtypeStruct((_M, 1), jnp.int32),
            jax.ShapeDtypeStruct((_NUM_CODES, _CODE_DIM), jnp.float32),
        ],
        scratch_shapes=[pltpu.VMEM((1, _NUM_CODES), jnp.float32)],
    )(z2d, codebook_w, proj_w, proj_b2d)


_NC, _NS = 2, 16
_NW = _NC * _NS
_BPW = _M // _NW
_CHUNK = 96
_NCHUNK = _BPW // _CHUNK


def _gather_body(table_hbm, idx_hbm, out_hbm, i0, i1, i2, rows_v, sem):
    wid = lax.axis_index("c") * _NS + lax.axis_index("s")
    base = wid * _BPW
    bufs = (i0, i1, i2)
    for c in range(_NCHUNK):
        pltpu.sync_copy(idx_hbm.at[pl.ds(base + c * _CHUNK, _CHUNK)], bufs[c])
    cps = [
        pltpu.async_copy(table_hbm.at[bufs[c]],
                         rows_v.at[pl.ds(c * _CHUNK, _CHUNK)], sem)
        for c in range(_NCHUNK)
    ]
    for cp in cps:
        cp.wait()
    pltpu.sync_copy(rows_v, out_hbm.at[pl.ds(base, _BPW)])


def _gather_rows(qc, idx_flat):
    mesh = plsc.VectorSubcoreMesh(core_axis_name="c", subcore_axis_name="s")
    f = pl.kernel(
        _gather_body,
        out_type=jax.ShapeDtypeStruct((_M, _CODE_DIM), jnp.float32),
        mesh=mesh,
        scratch_types=[
            pltpu.VMEM((_CHUNK,), jnp.int32),
            pltpu.VMEM((_CHUNK,), jnp.int32),
            pltpu.VMEM((_CHUNK,), jnp.int32),
            pltpu.VMEM((_BPW, _CODE_DIM), jnp.float32),
            pltpu.SemaphoreType.DMA,
        ],
    )
    return f(qc, idx_flat)


def kernel(z, codebook_w, proj_w, proj_b):
    z2d = z.reshape(-1, _CODE_DIM)
    idx2d, qc = _argmin_codes(
        z2d, codebook_w, proj_w, proj_b.reshape(1, _CODE_DIM))
    idx = idx2d.reshape(-1)
    z_q = _gather_rows(qc, idx)
    return z_q.reshape(z.shape), idx.reshape(z.shape[:-1])


# speedup vs baseline: 1.5192x; 1.0330x over previous
import functools

import jax
import jax.numpy as jnp
from jax import lax
from jax.experimental import pallas as pl
from jax.experimental.pallas import tpu as pltpu
from jax.experimental.pallas import tpu_sc as plsc

_NUM_CODES = 8192
_CODE_DIM = 256
_M = 9216


_ZBLK = 1024


_LANES = 128
_NCHUNKS = _NUM_CODES // _LANES


def _argmin_body(z_ref, cb_ref, pw_ref, pb_ref, idx_ref, qc_ref, cn_ref):
    @pl.when(pl.program_id(0) == 0)
    def _():
        qc = lax.dot_general(
            cb_ref[...], pw_ref[...], (((1,), (1,)), ((), ())),
            preferred_element_type=jnp.float32) + pb_ref[...]
        qc_ref[...] = qc
        cn_ref[...] = jnp.sum(qc * qc, axis=1, keepdims=True).reshape(
            1, _NUM_CODES)

    z = z_ref[...]
    s2 = lax.dot_general(
        z * -2.0, qc_ref[...], (((1,), (1,)), ((), ())),
        preferred_element_type=jnp.float32)
    zn = jnp.sum(z * z, axis=1, keepdims=True)
    cn = cn_ref[...]
    m_acc = jnp.full((_ZBLK, _LANES), jnp.float32(3.0e38))
    a_acc = jnp.zeros((_ZBLK, _LANES), jnp.int32)
    for g in range(_NCHUNKS):
        dg = (zn + cn[:, g * _LANES:(g + 1) * _LANES]) \
            + s2[:, g * _LANES:(g + 1) * _LANES]
        upd = dg < m_acc
        a_acc = jnp.where(upd, jnp.int32(g), a_acc)
        m_acc = jnp.minimum(m_acc, dg)
    bmin = jnp.min(m_acc, axis=1, keepdims=True)
    lane = lax.broadcasted_iota(jnp.int32, (_ZBLK, _LANES), 1)
    j = a_acc * _LANES + lane
    idx_ref[...] = jnp.min(
        jnp.where(m_acc == bmin, j, jnp.int32(1 << 30)),
        axis=1, keepdims=True)


def _argmin_codes(z2d, codebook_w, proj_w, proj_b2d):
    return pl.pallas_call(
        _argmin_body,
        grid=(_M // _ZBLK,),
        in_specs=[
            pl.BlockSpec((_ZBLK, _CODE_DIM), lambda i: (i, 0)),
            pl.BlockSpec((_NUM_CODES, _CODE_DIM), lambda i: (0, 0)),
            pl.BlockSpec((_CODE_DIM, _CODE_DIM), lambda i: (0, 0)),
            pl.BlockSpec((1, _CODE_DIM), lambda i: (0, 0)),
        ],
        out_specs=[
            pl.BlockSpec((_ZBLK, 1), lambda i: (i, 0)),
            pl.BlockSpec((_NUM_CODES, _CODE_DIM), lambda i: (0, 0)),
        ],
        out_shape=[
            jax.ShapeDtypeStruct((_M, 1), jnp.int32),
            jax.ShapeDtypeStruct((_NUM_CODES, _CODE_DIM), jnp.float32),
        ],
        scratch_shapes=[pltpu.VMEM((1, _NUM_CODES), jnp.float32)],
    )(z2d, codebook_w, proj_w, proj_b2d)


_NC, _NS = 2, 16
_NW = _NC * _NS
_BPW = _M // _NW
_CHUNK = 96
_NCHUNK = _BPW // _CHUNK


def _gather_body(table_hbm, idx_hbm, out_hbm, i0, i1, i2, rows_v, sem):
    wid = lax.axis_index("c") * _NS + lax.axis_index("s")
    base = wid * _BPW
    bufs = (i0, i1, i2)
    for c in range(_NCHUNK):
        pltpu.sync_copy(idx_hbm.at[pl.ds(base + c * _CHUNK, _CHUNK)], bufs[c])
    cps = [
        pltpu.async_copy(table_hbm.at[bufs[c]],
                         rows_v.at[pl.ds(c * _CHUNK, _CHUNK)], sem)
        for c in range(_NCHUNK)
    ]
    for cp in cps:
        cp.wait()
    pltpu.sync_copy(rows_v, out_hbm.at[pl.ds(base, _BPW)])


def _gather_rows(qc, idx_flat):
    mesh = plsc.VectorSubcoreMesh(core_axis_name="c", subcore_axis_name="s")
    f = pl.kernel(
        _gather_body,
        out_type=jax.ShapeDtypeStruct((_M, _CODE_DIM), jnp.float32),
        mesh=mesh,
        scratch_types=[
            pltpu.VMEM((_CHUNK,), jnp.int32),
            pltpu.VMEM((_CHUNK,), jnp.int32),
            pltpu.VMEM((_CHUNK,), jnp.int32),
            pltpu.VMEM((_BPW, _CODE_DIM), jnp.float32),
            pltpu.SemaphoreType.DMA,
        ],
    )
    return f(qc, idx_flat)


def kernel(z, codebook_w, proj_w, proj_b):
    z2d = z.reshape(-1, _CODE_DIM)
    idx2d, qc = _argmin_codes(
        z2d, codebook_w, proj_w, proj_b.reshape(1, _CODE_DIM))
    idx = idx2d.reshape(-1)
    z_q = _gather_rows(qc, idx)
    return z_q.reshape(z.shape), idx.reshape(z.shape[:-1])


# speedup vs baseline: 1.5277x; 1.0056x over previous
import functools

import jax
import jax.numpy as jnp
from jax import lax
from jax.experimental import pallas as pl
from jax.experimental.pallas import tpu as pltpu
from jax.experimental.pallas import tpu_sc as plsc

_NUM_CODES = 8192
_CODE_DIM = 256
_M = 9216


_ZBLK = 1536


_LANES = 128
_NCHUNKS = _NUM_CODES // _LANES


def _argmin_body(z_ref, cb_ref, pw_ref, pb_ref, idx_ref, qc_ref, cn_ref):
    @pl.when(pl.program_id(0) == 0)
    def _():
        qc = lax.dot_general(
            cb_ref[...], pw_ref[...], (((1,), (1,)), ((), ())),
            preferred_element_type=jnp.float32) + pb_ref[...]
        qc_ref[...] = qc
        cn_ref[...] = jnp.sum(qc * qc, axis=1, keepdims=True).reshape(
            1, _NUM_CODES)

    z = z_ref[...]
    s2 = lax.dot_general(
        z * -2.0, qc_ref[...], (((1,), (1,)), ((), ())),
        preferred_element_type=jnp.float32)
    zn = jnp.sum(z * z, axis=1, keepdims=True)
    cn = cn_ref[...]
    m_acc = jnp.full((_ZBLK, _LANES), jnp.float32(3.0e38))
    a_acc = jnp.zeros((_ZBLK, _LANES), jnp.int32)
    for g in range(_NCHUNKS):
        dg = (zn + cn[:, g * _LANES:(g + 1) * _LANES]) \
            + s2[:, g * _LANES:(g + 1) * _LANES]
        upd = dg < m_acc
        a_acc = jnp.where(upd, jnp.int32(g), a_acc)
        m_acc = jnp.minimum(m_acc, dg)
    bmin = jnp.min(m_acc, axis=1, keepdims=True)
    lane = lax.broadcasted_iota(jnp.int32, (_ZBLK, _LANES), 1)
    j = a_acc * _LANES + lane
    idx_ref[...] = jnp.min(
        jnp.where(m_acc == bmin, j, jnp.int32(1 << 30)),
        axis=1, keepdims=True)


def _argmin_codes(z2d, codebook_w, proj_w, proj_b2d):
    return pl.pallas_call(
        _argmin_body,
        grid=(_M // _ZBLK,),
        in_specs=[
            pl.BlockSpec((_ZBLK, _CODE_DIM), lambda i: (i, 0)),
            pl.BlockSpec((_NUM_CODES, _CODE_DIM), lambda i: (0, 0)),
            pl.BlockSpec((_CODE_DIM, _CODE_DIM), lambda i: (0, 0)),
            pl.BlockSpec((1, _CODE_DIM), lambda i: (0, 0)),
        ],
        out_specs=[
            pl.BlockSpec((_ZBLK, 1), lambda i: (i, 0)),
            pl.BlockSpec((_NUM_CODES, _CODE_DIM), lambda i: (0, 0)),
        ],
        out_shape=[
            jax.ShapeDtypeStruct((_M, 1), jnp.int32),
            jax.ShapeDtypeStruct((_NUM_CODES, _CODE_DIM), jnp.float32),
        ],
        scratch_shapes=[pltpu.VMEM((1, _NUM_CODES), jnp.float32)],
    )(z2d, codebook_w, proj_w, proj_b2d)


_NC, _NS = 2, 16
_NW = _NC * _NS
_BPW = _M // _NW
_CHUNK = 96
_NCHUNK = _BPW // _CHUNK


def _gather_body(table_hbm, idx_hbm, out_hbm, i0, i1, i2, rows_v, sem):
    wid = lax.axis_index("c") * _NS + lax.axis_index("s")
    base = wid * _BPW
    bufs = (i0, i1, i2)
    for c in range(_NCHUNK):
        pltpu.sync_copy(idx_hbm.at[pl.ds(base + c * _CHUNK, _CHUNK)], bufs[c])
    cps = [
        pltpu.async_copy(table_hbm.at[bufs[c]],
                         rows_v.at[pl.ds(c * _CHUNK, _CHUNK)], sem)
        for c in range(_NCHUNK)
    ]
    for cp in cps:
        cp.wait()
    pltpu.sync_copy(rows_v, out_hbm.at[pl.ds(base, _BPW)])


def _gather_rows(qc, idx_flat):
    mesh = plsc.VectorSubcoreMesh(core_axis_name="c", subcore_axis_name="s")
    f = pl.kernel(
        _gather_body,
        out_type=jax.ShapeDtypeStruct((_M, _CODE_DIM), jnp.float32),
        mesh=mesh,
        scratch_types=[
            pltpu.VMEM((_CHUNK,), jnp.int32),
            pltpu.VMEM((_CHUNK,), jnp.int32),
            pltpu.VMEM((_CHUNK,), jnp.int32),
            pltpu.VMEM((_BPW, _CODE_DIM), jnp.float32),
            pltpu.SemaphoreType.DMA,
        ],
    )
    return f(qc, idx_flat)


def kernel(z, codebook_w, proj_w, proj_b):
    z2d = z.reshape(-1, _CODE_DIM)
    idx2d, qc = _argmin_codes(
        z2d, codebook_w, proj_w, proj_b.reshape(1, _CODE_DIM))
    idx = idx2d.reshape(-1)
    z_q = _gather_rows(qc, idx)
    return z_q.reshape(z.shape), idx.reshape(z.shape[:-1])
